# baseline (speedup 1.0000x reference)
import functools

import jax
import jax.numpy as jnp
import numpy as np
from jax.experimental import pallas as pl

D_MODEL = 1024
H_TOTAL = 32
H_ACTIVE = 4
D_HEAD = D_MODEL // H_TOTAL
HG = 4
_BT = 256


def _router_body(x_ref, wg_ref, w_ref):
    logits = jnp.dot(x_ref[...], wg_ref[...],
                     preferred_element_type=jnp.float32)
    m = jnp.max(logits, axis=-1, keepdims=True)
    e = jnp.exp(logits - m)
    probs = e / jnp.sum(e, axis=-1, keepdims=True)
    col = jax.lax.broadcasted_iota(jnp.int32, probs.shape, 1)
    p = probs
    w = jnp.zeros_like(probs)
    for _ in range(H_ACTIVE):
        mx = jnp.max(p, axis=-1, keepdims=True)
        cand = jnp.where(p == mx, col, H_TOTAL)
        first = jnp.min(cand, axis=-1, keepdims=True)
        sel = col == first
        w = jnp.where(sel, probs, w)
        p = jnp.where(sel, -jnp.inf, p)
    w_ref[...] = w


def _matmul_body(x_ref, w_ref, o_ref):
    o_ref[...] = jnp.dot(x_ref[...], w_ref[...],
                         preferred_element_type=jnp.float32)


def _qkv_body(x_ref, w_ref, o_ref):
    o_ref[...] = jnp.dot(x_ref[...], w_ref[...],
                         preferred_element_type=jnp.float32
                         ).astype(jnp.bfloat16)


def _attn_body(q_ref, k_ref, v_ref, g_ref, o_ref, *, scale):
    g = pl.program_id(0)
    i = pl.program_id(1)
    q_blk = q_ref[...]
    sel = (jax.lax.broadcasted_iota(jnp.int32, (H_TOTAL, HG), 0)
           == g * HG + jax.lax.broadcasted_iota(jnp.int32, (H_TOTAL, HG), 1)
           ).astype(jnp.float32)
    gates = jnp.dot(g_ref[...], sel,
                    preferred_element_type=jnp.float32)

    row = i * _BT + jax.lax.broadcasted_iota(jnp.int32, (_BT, _BT), 0)
    colb = jax.lax.broadcasted_iota(jnp.int32, (_BT, _BT), 1)

    for hh in range(HG):
        q = q_blk[:, hh * D_HEAD:(hh + 1) * D_HEAD]

        def body(j, carry):
            m, l, acc = carry
            kj = k_ref[pl.ds(j * _BT, _BT), hh * D_HEAD:(hh + 1) * D_HEAD]
            vj = v_ref[pl.ds(j * _BT, _BT), hh * D_HEAD:(hh + 1) * D_HEAD]
            s = jax.lax.dot_general(q, kj, (((1,), (1,)), ((), ())),
                                    preferred_element_type=jnp.float32)
            s = s * scale
            s = jnp.where(j * _BT + colb <= row, s, jnp.float32(-1e9))
            m_new = jnp.maximum(m, jnp.max(s, axis=-1, keepdims=True))
            p = jnp.exp(s - m_new)
            alpha = jnp.exp(m - m_new)
            l_new = l * alpha + jnp.sum(p, axis=-1, keepdims=True)
            acc_new = acc * alpha + jnp.dot(
                p.astype(jnp.bfloat16), vj,
                preferred_element_type=jnp.float32)
            return m_new, l_new, acc_new

        m0 = jnp.full((_BT, 1), -jnp.inf, jnp.float32)
        l0 = jnp.zeros((_BT, 1), jnp.float32)
        a0 = jnp.zeros((_BT, D_HEAD), jnp.float32)
        m, l, acc = jax.lax.fori_loop(0, i + 1, body, (m0, l0, a0))
        out = (acc / l) * gates[:, hh:hh + 1]
        o_ref[:, hh * D_HEAD:(hh + 1) * D_HEAD] = out


@jax.jit
def kernel(x, Wg, Wq, Wk, Wv, Wo):
    b, t, d = x.shape
    x2 = x.reshape(t, d)

    w = pl.pallas_call(
        _router_body,
        grid=(t // _BT,),
        in_specs=[
            pl.BlockSpec((_BT, d), lambda i: (i, 0)),
            pl.BlockSpec((d, H_TOTAL), lambda i: (0, 0)),
        ],
        out_specs=pl.BlockSpec((_BT, H_TOTAL), lambda i: (i, 0)),
        out_shape=jax.ShapeDtypeStruct((t, H_TOTAL), jnp.float32),
    )(x2, Wg)

    xb = x2.astype(jnp.bfloat16)
    wqkv = jnp.concatenate([Wq, Wk, Wv], axis=1).astype(jnp.bfloat16)
    bn = 512
    qkv = pl.pallas_call(
        _qkv_body,
        grid=(3 * d // bn,),
        in_specs=[
            pl.BlockSpec((t, d), lambda j: (0, 0)),
            pl.BlockSpec((d, bn), lambda j: (0, j)),
        ],
        out_specs=pl.BlockSpec((t, bn), lambda j: (0, j)),
        out_shape=jax.ShapeDtypeStruct((t, 3 * d), jnp.bfloat16),
    )(xb, wqkv)

    scale = np.float32(1.0 / np.sqrt(D_HEAD))
    ng = H_TOTAL // HG
    wide = HG * D_HEAD
    attn_out = pl.pallas_call(
        functools.partial(_attn_body, scale=scale),
        grid=(ng, t // _BT),
        in_specs=[
            pl.BlockSpec((_BT, wide), lambda g, i: (i, g)),
            pl.BlockSpec((t, wide), lambda g, i: (0, ng + g)),
            pl.BlockSpec((t, wide), lambda g, i: (0, 2 * ng + g)),
            pl.BlockSpec((_BT, H_TOTAL), lambda g, i: (i, 0)),
        ],
        out_specs=pl.BlockSpec((_BT, wide), lambda g, i: (i, g)),
        out_shape=jax.ShapeDtypeStruct((t, d), jnp.float32),
    )(qkv, qkv, qkv, w)

    y = pl.pallas_call(
        _matmul_body,
        grid=(d // bn,),
        in_specs=[
            pl.BlockSpec((t, d), lambda j: (0, 0)),
            pl.BlockSpec((d, bn), lambda j: (0, j)),
        ],
        out_specs=pl.BlockSpec((t, bn), lambda j: (0, j)),
        out_shape=jax.ShapeDtypeStruct((t, d), jnp.float32),
    )(attn_out.astype(jnp.bfloat16), Wo.astype(jnp.bfloat16))

    return y.reshape(b, t, d)


# speedup vs baseline: 2.1507x; 2.1507x over previous
import functools

import jax
import jax.numpy as jnp
import numpy as np
from jax.experimental import pallas as pl

D_MODEL = 1024
H_TOTAL = 32
H_ACTIVE = 4
D_HEAD = D_MODEL // H_TOTAL
HG = 4
_BT = 256


def _router_body(x_ref, wg_ref, w_ref):
    logits = jnp.dot(x_ref[...], wg_ref[...],
                     preferred_element_type=jnp.float32)
    m = jnp.max(logits, axis=-1, keepdims=True)
    e = jnp.exp(logits - m)
    probs = e / jnp.sum(e, axis=-1, keepdims=True)
    col = jax.lax.broadcasted_iota(jnp.int32, probs.shape, 1)
    p = probs
    w = jnp.zeros_like(probs)
    for _ in range(H_ACTIVE):
        mx = jnp.max(p, axis=-1, keepdims=True)
        cand = jnp.where(p == mx, col, H_TOTAL)
        first = jnp.min(cand, axis=-1, keepdims=True)
        sel = col == first
        w = jnp.where(sel, probs, w)
        p = jnp.where(sel, -jnp.inf, p)
    w_ref[...] = w


def _matmul_body(x_ref, w_ref, o_ref):
    o_ref[...] = jnp.dot(x_ref[...], w_ref[...],
                         preferred_element_type=jnp.float32)


def _qkv_body(x_ref, w_ref, o_ref):
    o_ref[...] = jnp.dot(x_ref[...], w_ref[...],
                         preferred_element_type=jnp.float32
                         ).astype(jnp.bfloat16)


def _attn_body(q_ref, k_ref, v_ref, g_ref, o_ref, *, scale, t):
    g = pl.program_id(0)
    i = pl.program_id(1)
    q_blk = q_ref[...]
    sel = (jax.lax.broadcasted_iota(jnp.int32, (H_TOTAL, HG), 0)
           == g * HG + jax.lax.broadcasted_iota(jnp.int32, (H_TOTAL, HG), 1)
           ).astype(jnp.float32)
    gates = jnp.dot(g_ref[...], sel,
                    preferred_element_type=jnp.float32)

    row = i * _BT + jax.lax.broadcasted_iota(jnp.int32, (_BT, t), 0)
    col = jax.lax.broadcasted_iota(jnp.int32, (_BT, t), 1)
    neg = jnp.float32(-1e9)

    for hh in range(HG):
        q = q_blk[:, hh * D_HEAD:(hh + 1) * D_HEAD]
        k = k_ref[:, hh * D_HEAD:(hh + 1) * D_HEAD]
        s = jax.lax.dot_general(q, k, (((1,), (1,)), ((), ())),
                                preferred_element_type=jnp.float32)
        s = jnp.where(col <= row, s * scale, neg)
        m = jnp.max(s, axis=-1, keepdims=True)
        p = jnp.exp(s - m)
        l = jnp.sum(p, axis=-1, keepdims=True)
        out = jnp.dot(p.astype(jnp.bfloat16),
                      v_ref[:, hh * D_HEAD:(hh + 1) * D_HEAD],
                      preferred_element_type=jnp.float32)
        o_ref[:, hh * D_HEAD:(hh + 1) * D_HEAD] = \
            (out / l) * gates[:, hh:hh + 1]


@jax.jit
def kernel(x, Wg, Wq, Wk, Wv, Wo):
    b, t, d = x.shape
    x2 = x.reshape(t, d)

    w = pl.pallas_call(
        _router_body,
        grid=(t // _BT,),
        in_specs=[
            pl.BlockSpec((_BT, d), lambda i: (i, 0)),
            pl.BlockSpec((d, H_TOTAL), lambda i: (0, 0)),
        ],
        out_specs=pl.BlockSpec((_BT, H_TOTAL), lambda i: (i, 0)),
        out_shape=jax.ShapeDtypeStruct((t, H_TOTAL), jnp.float32),
    )(x2, Wg)

    xb = x2.astype(jnp.bfloat16)
    wqkv = jnp.concatenate([Wq, Wk, Wv], axis=1).astype(jnp.bfloat16)
    bn = 512
    qkv = pl.pallas_call(
        _qkv_body,
        grid=(3 * d // bn,),
        in_specs=[
            pl.BlockSpec((t, d), lambda j: (0, 0)),
            pl.BlockSpec((d, bn), lambda j: (0, j)),
        ],
        out_specs=pl.BlockSpec((t, bn), lambda j: (0, j)),
        out_shape=jax.ShapeDtypeStruct((t, 3 * d), jnp.bfloat16),
    )(xb, wqkv)

    scale = np.float32(1.0 / np.sqrt(D_HEAD))
    ng = H_TOTAL // HG
    wide = HG * D_HEAD
    attn_out = pl.pallas_call(
        functools.partial(_attn_body, scale=scale, t=t),
        grid=(ng, t // _BT),
        in_specs=[
            pl.BlockSpec((_BT, wide), lambda g, i: (i, g)),
            pl.BlockSpec((t, wide), lambda g, i: (0, ng + g)),
            pl.BlockSpec((t, wide), lambda g, i: (0, 2 * ng + g)),
            pl.BlockSpec((_BT, H_TOTAL), lambda g, i: (i, 0)),
        ],
        out_specs=pl.BlockSpec((_BT, wide), lambda g, i: (i, g)),
        out_shape=jax.ShapeDtypeStruct((t, d), jnp.float32),
    )(qkv, qkv, qkv, w)

    y = pl.pallas_call(
        _matmul_body,
        grid=(d // bn,),
        in_specs=[
            pl.BlockSpec((t, d), lambda j: (0, 0)),
            pl.BlockSpec((d, bn), lambda j: (0, j)),
        ],
        out_specs=pl.BlockSpec((t, bn), lambda j: (0, j)),
        out_shape=jax.ShapeDtypeStruct((t, d), jnp.float32),
    )(attn_out.astype(jnp.bfloat16), Wo.astype(jnp.bfloat16))

    return y.reshape(b, t, d)
